# baseline (speedup 1.0000x reference)
import jax
import jax.numpy as jnp
from jax.experimental import pallas as pl
from jax.experimental.pallas import tpu as pltpu

_K = 1024
_D = 256
_RB = 2304


def _vq_body(x_ref, cb_ref, out_ref, c2_ref):
    @pl.when(pl.program_id(0) == 0)
    def _init():
        cb0 = cb_ref[...]
        c2_col = jnp.sum(cb0 * cb0, axis=1, keepdims=True)
        c2_ref[...] = jax.lax.transpose(c2_col, (1, 0))

    xs = -2.0 * x_ref[...]
    cb = cb_ref[...]
    mm2 = jax.lax.dot_general(
        xs, cb, (((1,), (1,)), ((), ())),
        preferred_element_type=jnp.float32,
    )
    x_sqr = 0.25 * jnp.sum(xs * xs, axis=1, keepdims=True)
    dist = (c2_ref[...] + x_sqr) + mm2
    m = jnp.min(dist, axis=1, keepdims=True)
    iota = jax.lax.broadcasted_iota(jnp.int32, (1, _K), 1).astype(jnp.float32)
    cand = jnp.where(dist == m, iota, float(_K))
    idx = jnp.min(cand, axis=1, keepdims=True)
    out_ref[...] = idx.astype(jnp.int32)


@jax.jit
def kernel(z_e_x, codebook):
    b, c, h, w = z_e_x.shape
    n = b * h * w
    x = jnp.transpose(z_e_x, (0, 2, 3, 1)).reshape(n, c)
    out = pl.pallas_call(
        _vq_body,
        grid=(n // _RB,),
        in_specs=[
            pl.BlockSpec((_RB, _D), lambda i: (i, 0)),
            pl.BlockSpec((_K, _D), lambda i: (0, 0)),
        ],
        out_specs=pl.BlockSpec((_RB, 1), lambda i: (i, 0)),
        out_shape=jax.ShapeDtypeStruct((n, 1), jnp.int32),
        scratch_shapes=[pltpu.VMEM((1, _K), jnp.float32)],
    )(x, codebook)
    return out.reshape(b, h, w)


# speedup vs baseline: 1.0415x; 1.0415x over previous
import jax
import jax.numpy as jnp
from jax.experimental import pallas as pl
from jax.experimental.pallas import tpu as pltpu

_K = 1024
_D = 256
_RB = 4608


def _vq_body(x_ref, cb_ref, out_ref, c2_ref):
    @pl.when(pl.program_id(0) == 0)
    def _init():
        cb0 = cb_ref[...]
        c2_col = jnp.sum(cb0 * cb0, axis=1, keepdims=True)
        c2_ref[...] = jax.lax.transpose(c2_col, (1, 0))

    xs = -2.0 * x_ref[...]
    cb = cb_ref[...]
    mm2 = jax.lax.dot_general(
        xs, cb, (((1,), (1,)), ((), ())),
        preferred_element_type=jnp.float32,
    )
    x_sqr = 0.25 * jnp.sum(xs * xs, axis=1, keepdims=True)
    dist = (c2_ref[...] + x_sqr) + mm2
    m = jnp.min(dist, axis=1, keepdims=True)
    iota = jax.lax.broadcasted_iota(jnp.int32, (1, _K), 1).astype(jnp.float32)
    cand = jnp.where(dist == m, iota, float(_K))
    idx = jnp.min(cand, axis=1, keepdims=True)
    out_ref[...] = idx.astype(jnp.int32)


@jax.jit
def kernel(z_e_x, codebook):
    b, c, h, w = z_e_x.shape
    n = b * h * w
    x = jnp.transpose(z_e_x, (0, 2, 3, 1)).reshape(n, c)
    out = pl.pallas_call(
        _vq_body,
        grid=(n // _RB,),
        in_specs=[
            pl.BlockSpec((_RB, _D), lambda i: (i, 0)),
            pl.BlockSpec((_K, _D), lambda i: (0, 0)),
        ],
        out_specs=pl.BlockSpec((_RB, 1), lambda i: (i, 0)),
        out_shape=jax.ShapeDtypeStruct((n, 1), jnp.int32),
        scratch_shapes=[pltpu.VMEM((1, _K), jnp.float32)],
    )(x, codebook)
    return out.reshape(b, h, w)
